# baseline (speedup 1.0000x reference)
import jax
import jax.numpy as jnp
import numpy as np
from jax.experimental import pallas as pl
from jax.experimental.pallas import tpu as pltpu

ROWS = 128
COLS = 100000
BR = 8
BC = 1024
NCB = (COLS + BC - 1) // BC

_U32 = jnp.uint32
_TINY = np.float32(np.finfo(np.float32).tiny)
_NEG_INF = np.float32(-np.inf)


def _threefry_xor_fold(x1):
    k0 = np.uint32(0)
    k1 = np.uint32(42)
    ks = (k0, k1, np.uint32(k0 ^ k1 ^ np.uint32(0x1BD11BDA)))
    rot = ((13, 15, 26, 6), (17, 29, 16, 24))

    x0 = jnp.zeros_like(x1)
    for n in range(5):
        for r in rot[n % 2]:
            x0 = x0 + x1
            x1 = (x1 << _U32(r)) | (x1 >> _U32(32 - r))
            x1 = x1 ^ x0
        x0 = x0 + ks[(n + 1) % 3]
        x1 = x1 + ks[(n + 2) % 3] + _U32(n + 1)
    return x0 ^ x1


def _gumbel_from_bits(bits):
    fl = jax.lax.bitcast_convert_type(
        (bits >> _U32(9)) | _U32(0x3F800000), jnp.float32
    )
    u = (fl - np.float32(1.0)) + _TINY
    return -jnp.log(-jnp.log(u))


def _body(x_ref, out_ref, bestv, besti):
    r = pl.program_id(0)
    c = pl.program_id(1)

    lane = jax.lax.broadcasted_iota(jnp.int32, (BR, BC), 1)
    row = r * BR + jax.lax.broadcasted_iota(jnp.int32, (BR, BC), 0)
    col = c * BC + lane
    x1 = (row * COLS + col + 42).astype(_U32)

    g = _gumbel_from_bits(_threefry_xor_fold(x1))
    val = x_ref[...] + g
    val = jnp.where(col < COLS, val, _NEG_INF)

    @pl.when(c == 0)
    def _init():
        bestv[...] = jnp.full((BR, BC), _NEG_INF, jnp.float32)
        besti[...] = jnp.zeros((BR, BC), jnp.int32)

    bv = bestv[...]
    bi = besti[...]
    upd = val > bv
    bv = jnp.where(upd, val, bv)
    bi = jnp.where(upd, col, bi)
    bestv[...] = bv
    besti[...] = bi

    @pl.when(c == NCB - 1)
    def _emit():
        m = jnp.max(bv, axis=1, keepdims=True)
        cand = jnp.where(bv == m, bi, jnp.int32(COLS))
        out_ref[...] = jnp.min(cand, axis=1, keepdims=True)


@jax.jit
def _run(outputs):
    out = pl.pallas_call(
        _body,
        grid=(ROWS // BR, NCB),
        in_specs=[pl.BlockSpec((BR, BC), lambda r, c: (r, c))],
        out_specs=pl.BlockSpec((BR, 1), lambda r, c: (r, 0)),
        out_shape=jax.ShapeDtypeStruct((ROWS, 1), jnp.int32),
        scratch_shapes=[
            pltpu.VMEM((BR, BC), jnp.float32),
            pltpu.VMEM((BR, BC), jnp.int32),
        ],
        compiler_params=pltpu.CompilerParams(
            dimension_semantics=("parallel", "arbitrary"),
        ),
    )(outputs)
    return out[:, 0]


def kernel(outputs):
    return _run(outputs)


# speedup vs baseline: 2.0921x; 2.0921x over previous
import jax
import jax.numpy as jnp
import numpy as np
from jax.experimental import pallas as pl
from jax.experimental.pallas import tpu as pltpu

ROWS = 128
COLS = 100000
BR = 8
BC = 8192
NCB = (COLS + BC - 1) // BC

_U32 = jnp.uint32
_TINY = np.float32(np.finfo(np.float32).tiny)
_NEG_INF = np.float32(-np.inf)


def _threefry_xor_fold(x1):
    k0 = np.uint32(0)
    k1 = np.uint32(42)
    ks = (k0, k1, np.uint32(k0 ^ k1 ^ np.uint32(0x1BD11BDA)))
    rot = ((13, 15, 26, 6), (17, 29, 16, 24))

    x0 = jnp.zeros_like(x1)
    for n in range(5):
        for r in rot[n % 2]:
            x0 = x0 + x1
            x1 = (x1 << _U32(r)) | (x1 >> _U32(32 - r))
            x1 = x1 ^ x0
        x0 = x0 + ks[(n + 1) % 3]
        x1 = x1 + ks[(n + 2) % 3] + _U32(n + 1)
    return x0 ^ x1


def _gumbel_from_bits(bits):
    fl = jax.lax.bitcast_convert_type(
        (bits >> _U32(9)) | _U32(0x3F800000), jnp.float32
    )
    u = (fl - np.float32(1.0)) + _TINY
    return -jnp.log(-jnp.log(u))


def _body(x_ref, out_ref, bestv, besti):
    r = pl.program_id(0)
    c = pl.program_id(1)

    lane = jax.lax.broadcasted_iota(jnp.int32, (BR, BC), 1)
    row = r * BR + jax.lax.broadcasted_iota(jnp.int32, (BR, BC), 0)
    col = c * BC + lane
    x1 = (row * COLS + col + 42).astype(_U32)

    g = _gumbel_from_bits(_threefry_xor_fold(x1))
    val = x_ref[...] + g
    val = jnp.where(col < COLS, val, _NEG_INF)

    @pl.when(c == 0)
    def _init():
        bestv[...] = jnp.full((BR, BC), _NEG_INF, jnp.float32)
        besti[...] = jnp.zeros((BR, BC), jnp.int32)

    bv = bestv[...]
    bi = besti[...]
    upd = val > bv
    bv = jnp.where(upd, val, bv)
    bi = jnp.where(upd, col, bi)
    bestv[...] = bv
    besti[...] = bi

    @pl.when(c == NCB - 1)
    def _emit():
        m = jnp.max(bv, axis=1, keepdims=True)
        cand = jnp.where(bv == m, bi, jnp.int32(COLS))
        out_ref[...] = jnp.min(cand, axis=1, keepdims=True)


@jax.jit
def _run(outputs):
    out = pl.pallas_call(
        _body,
        grid=(ROWS // BR, NCB),
        in_specs=[pl.BlockSpec((BR, BC), lambda r, c: (r, c))],
        out_specs=pl.BlockSpec((BR, 1), lambda r, c: (r, 0)),
        out_shape=jax.ShapeDtypeStruct((ROWS, 1), jnp.int32),
        scratch_shapes=[
            pltpu.VMEM((BR, BC), jnp.float32),
            pltpu.VMEM((BR, BC), jnp.int32),
        ],
        compiler_params=pltpu.CompilerParams(
            dimension_semantics=("parallel", "arbitrary"),
        ),
    )(outputs)
    return out[:, 0]


def kernel(outputs):
    return _run(outputs)


# speedup vs baseline: 2.8279x; 1.3517x over previous
import jax
import jax.numpy as jnp
import numpy as np
from jax.experimental import pallas as pl
from jax.experimental.pallas import tpu as pltpu

ROWS = 128
COLS = 100000
BR = 8
BC = 8192
NCB = (COLS + BC - 1) // BC

_U32 = jnp.uint32
_TINY = np.float32(np.finfo(np.float32).tiny)
_NEG_INF = np.float32(-np.inf)


def _threefry_xor_fold(x1):
    k0 = np.uint32(0)
    k1 = np.uint32(42)
    ks = (k0, k1, np.uint32(k0 ^ k1 ^ np.uint32(0x1BD11BDA)))
    rot = ((13, 15, 26, 6), (17, 29, 16, 24))

    x0 = jnp.zeros_like(x1)
    for n in range(5):
        for r in rot[n % 2]:
            x0 = x0 + x1
            x1 = (x1 << _U32(r)) | (x1 >> _U32(32 - r))
            x1 = x1 ^ x0
        x0 = x0 + ks[(n + 1) % 3]
        x1 = x1 + ks[(n + 2) % 3] + _U32(n + 1)
    return x0 ^ x1


def _gumbel_from_bits(bits):
    fl = jax.lax.bitcast_convert_type(
        (bits >> _U32(9)) | _U32(0x3F800000), jnp.float32
    )
    u = (fl - np.float32(1.0)) + _TINY
    return -jnp.log(-jnp.log(u))


CH = 128
NCH = BC // CH


def _body(x_ref, out_ref, bestv, besti):
    r = pl.program_id(0)
    c = pl.program_id(1)

    lane = jax.lax.broadcasted_iota(jnp.int32, (BR, CH), 1)
    row = r * BR + jax.lax.broadcasted_iota(jnp.int32, (BR, CH), 0)
    base42 = row * COLS + lane + 42
    cbase = c * BC

    bv = jnp.full((BR, CH), _NEG_INF, jnp.float32)
    bi = jnp.zeros((BR, CH), jnp.int32)
    for j in range(NCH):
        x1 = (base42 + (cbase + j * CH)).astype(_U32)
        g = _gumbel_from_bits(_threefry_xor_fold(x1))
        val = x_ref[:, j * CH:(j + 1) * CH] + g
        col = lane + (cbase + j * CH)
        val = jnp.where(col < COLS, val, _NEG_INF)
        upd = val > bv
        bv = jnp.where(upd, val, bv)
        bi = jnp.where(upd, col, bi)

    @pl.when(c == 0)
    def _init():
        bestv[...] = bv
        besti[...] = bi

    @pl.when(c != 0)
    def _fold():
        ov = bestv[...]
        oi = besti[...]
        upd = bv > ov
        nv = jnp.where(upd, bv, ov)
        bestv[...] = nv
        besti[...] = jnp.where(upd, bi, oi)

        @pl.when(c == NCB - 1)
        def _emit():
            m = jnp.max(nv, axis=1, keepdims=True)
            ni = jnp.where(upd, bi, oi)
            cand = jnp.where(nv == m, ni, jnp.int32(COLS))
            out_ref[...] = jnp.min(cand, axis=1, keepdims=True)


@jax.jit
def _run(outputs):
    out = pl.pallas_call(
        _body,
        grid=(ROWS // BR, NCB),
        in_specs=[pl.BlockSpec((BR, BC), lambda r, c: (r, c))],
        out_specs=pl.BlockSpec((BR, 1), lambda r, c: (r, 0)),
        out_shape=jax.ShapeDtypeStruct((ROWS, 1), jnp.int32),
        scratch_shapes=[
            pltpu.VMEM((BR, CH), jnp.float32),
            pltpu.VMEM((BR, CH), jnp.int32),
        ],
        compiler_params=pltpu.CompilerParams(
            dimension_semantics=("parallel", "arbitrary"),
        ),
    )(outputs)
    return out[:, 0]


def kernel(outputs):
    return _run(outputs)
